# baseline (speedup 1.0000x reference)
import jax
import jax.numpy as jnp
from jax.experimental import pallas as pl
from jax.experimental.pallas import tpu as pltpu

_CHUNKS = 8


def _dma_body(pos_ref, cache_ref, update_ref, out_ref, copy_sem, upd_sem):
    b = cache_ref.shape[0]
    per = b // _CHUNKS
    copies = [
        pltpu.make_async_copy(
            cache_ref.at[pl.ds(i * per, per)],
            out_ref.at[pl.ds(i * per, per)],
            copy_sem,
        )
        for i in range(_CHUNKS)
    ]
    for c in copies:
        c.start()
    for c in copies:
        c.wait()
    p = pos_ref[0]
    upd_len = update_ref.shape[2]
    upd = pltpu.make_async_copy(
        update_ref, out_ref.at[:, :, pl.ds(p, upd_len), :], upd_sem
    )
    upd.start()
    upd.wait()


def kernel(cache, update, pos):
    pos_arr = jnp.asarray(pos, jnp.int32).reshape((1,))
    return pl.pallas_call(
        _dma_body,
        out_shape=jax.ShapeDtypeStruct(cache.shape, cache.dtype),
        in_specs=[
            pl.BlockSpec(memory_space=pltpu.SMEM),
            pl.BlockSpec(memory_space=pltpu.MemorySpace.HBM),
            pl.BlockSpec(memory_space=pltpu.MemorySpace.HBM),
        ],
        out_specs=pl.BlockSpec(memory_space=pltpu.MemorySpace.HBM),
        scratch_shapes=[pltpu.SemaphoreType.DMA, pltpu.SemaphoreType.DMA],
    )(pos_arr, cache, update)


# speedup vs baseline: 20.1150x; 20.1150x over previous
import functools

import jax
import jax.numpy as jnp
from jax.experimental import pallas as pl
from jax.experimental.pallas import tpu as pltpu

_SEQ_BLK = 1024


def _copy_body(pos_ref, cache_ref, update_ref, out_ref):
    j = pl.program_id(1)
    pos = pos_ref[0]
    row0 = j * _SEQ_BLK
    upd_len = update_ref.shape[1]
    d = cache_ref.shape[2]

    rel = (
        jax.lax.broadcasted_iota(jnp.int32, (_SEQ_BLK, upd_len), 0)
        + row0
        - pos
    )
    k_iota = jax.lax.broadcasted_iota(jnp.int32, (_SEQ_BLK, upd_len), 1)
    onehot = (rel == k_iota).astype(jnp.float32)
    shifted = jax.lax.dot_general(
        onehot,
        update_ref[0],
        (((1,), (0,)), ((), ())),
        preferred_element_type=jnp.float32,
    )

    row_rel = (
        jax.lax.broadcasted_iota(jnp.int32, (_SEQ_BLK, d), 0) + row0 - pos
    )
    in_upd = (row_rel >= 0) & (row_rel < upd_len)
    out_ref[0] = jnp.where(in_upd, shifted, cache_ref[0])


def kernel(cache, update, pos):
    b, h, s, d = cache.shape
    upd_len = update.shape[-2]
    cache3 = cache.reshape(b * h, s, d)
    update3 = update.reshape(b * h, upd_len, d)
    pos_arr = jnp.asarray(pos, jnp.int32).reshape((1,))

    grid = (b * h, s // _SEQ_BLK)
    out3 = pl.pallas_call(
        _copy_body,
        grid_spec=pltpu.PrefetchScalarGridSpec(
            num_scalar_prefetch=1,
            grid=grid,
            in_specs=[
                pl.BlockSpec((1, _SEQ_BLK, d), lambda i, j, pos_ref: (i, j, 0)),
                pl.BlockSpec((1, upd_len, d), lambda i, j, pos_ref: (i, 0, 0)),
            ],
            out_specs=pl.BlockSpec(
                (1, _SEQ_BLK, d), lambda i, j, pos_ref: (i, j, 0)
            ),
        ),
        out_shape=jax.ShapeDtypeStruct((b * h, s, d), cache.dtype),
    )(pos_arr, cache3, update3)
    return out3.reshape(b, h, s, d)


# speedup vs baseline: 48.4272x; 2.4075x over previous
import jax
import jax.numpy as jnp
from jax.experimental import pallas as pl
from jax.experimental.pallas import tpu as pltpu

_SEQ_BLK = 4096
_BH_BLK = 2


def _copy_body(pos_ref, cache_ref, update_ref, out_ref):
    j = pl.program_id(1)
    pos = pos_ref[0]
    row0 = j * _SEQ_BLK
    upd_len = update_ref.shape[1]
    d = cache_ref.shape[2]

    out_ref[...] = cache_ref[...]

    overlaps = (pos < row0 + _SEQ_BLK) & (row0 < pos + upd_len)

    @pl.when(overlaps)
    def _patch():
        win = jnp.clip(pos - row0, 0, _SEQ_BLK - upd_len)
        rel = (
            jax.lax.broadcasted_iota(jnp.int32, (upd_len, d), 0)
            + row0
            + win
            - pos
        )
        for b in range(_BH_BLK):
            val = out_ref[b, pl.ds(win, upd_len), :]
            for k in range(upd_len):
                val = jnp.where(rel == k, update_ref[b, k, :][None, :], val)
            out_ref[b, pl.ds(win, upd_len), :] = val


def kernel(cache, update, pos):
    b, h, s, d = cache.shape
    upd_len = update.shape[-2]
    cache3 = cache.reshape(b * h, s, d)
    update3 = update.reshape(b * h, upd_len, d)
    pos_arr = jnp.asarray(pos, jnp.int32).reshape((1,))

    grid = (b * h // _BH_BLK, s // _SEQ_BLK)
    out3 = pl.pallas_call(
        _copy_body,
        grid_spec=pltpu.PrefetchScalarGridSpec(
            num_scalar_prefetch=1,
            grid=grid,
            in_specs=[
                pl.BlockSpec(
                    (_BH_BLK, _SEQ_BLK, d), lambda i, j, pos_ref: (i, j, 0)
                ),
                pl.BlockSpec(
                    (_BH_BLK, upd_len, d), lambda i, j, pos_ref: (i, 0, 0)
                ),
            ],
            out_specs=pl.BlockSpec(
                (_BH_BLK, _SEQ_BLK, d), lambda i, j, pos_ref: (i, j, 0)
            ),
        ),
        out_shape=jax.ShapeDtypeStruct((b * h, s, d), cache.dtype),
    )(pos_arr, cache3, update3)
    return out3.reshape(b, h, s, d)


# speedup vs baseline: 48.9913x; 1.0116x over previous
import jax
import jax.numpy as jnp
from jax.experimental import pallas as pl
from jax.experimental.pallas import tpu as pltpu

_SEQ_BLK = 4096
_BH_BLK = 4


def _copy_body(pos_ref, cache_ref, update_ref, out_ref):
    j = pl.program_id(1)
    pos = pos_ref[0]
    row0 = j * _SEQ_BLK
    upd_len = update_ref.shape[1]
    d = cache_ref.shape[2]

    out_ref[...] = cache_ref[...]

    overlaps = (pos < row0 + _SEQ_BLK) & (row0 < pos + upd_len)

    @pl.when(overlaps)
    def _patch():
        win = jnp.clip(pos - row0, 0, _SEQ_BLK - upd_len)
        rel = (
            jax.lax.broadcasted_iota(jnp.int32, (upd_len, d), 0)
            + row0
            + win
            - pos
        )
        for b in range(_BH_BLK):
            val = out_ref[b, pl.ds(win, upd_len), :]
            for k in range(upd_len):
                val = jnp.where(rel == k, update_ref[b, k, :][None, :], val)
            out_ref[b, pl.ds(win, upd_len), :] = val


def kernel(cache, update, pos):
    b, h, s, d = cache.shape
    upd_len = update.shape[-2]
    cache3 = cache.reshape(b * h, s, d)
    update3 = update.reshape(b * h, upd_len, d)
    pos_arr = jnp.asarray(pos, jnp.int32).reshape((1,))

    grid = (b * h // _BH_BLK, s // _SEQ_BLK)
    out3 = pl.pallas_call(
        _copy_body,
        grid_spec=pltpu.PrefetchScalarGridSpec(
            num_scalar_prefetch=1,
            grid=grid,
            in_specs=[
                pl.BlockSpec(
                    (_BH_BLK, _SEQ_BLK, d), lambda i, j, pos_ref: (i, j, 0)
                ),
                pl.BlockSpec(
                    (_BH_BLK, upd_len, d), lambda i, j, pos_ref: (i, 0, 0)
                ),
            ],
            out_specs=pl.BlockSpec(
                (_BH_BLK, _SEQ_BLK, d), lambda i, j, pos_ref: (i, j, 0)
            ),
        ),
        out_shape=jax.ShapeDtypeStruct((b * h, s, d), cache.dtype),
    )(pos_arr, cache3, update3)
    return out3.reshape(b, h, s, d)
